# baseline (speedup 1.0000x reference)
import functools

import jax
import jax.numpy as jnp
from jax import lax
from jax.experimental import pallas as pl
from jax.experimental.pallas import tpu as pltpu
from jax.experimental.pallas import tpu_sc as plsc

NC = 2
NS = 16
LANES = 16
CH = 80


def _mesh():
    return plsc.VectorSubcoreMesh(core_axis_name="c", subcore_axis_name="s")


def _npad(n_nodes):
    return -(-n_nodes // (NS * CH)) * (NS * CH)


def _zero_fill(buf, rows_n, width):
    zero16 = jnp.zeros((LANES,), jnp.float32)

    def zstore(t, carry):
        r = t // (width // LANES)
        c = (t % (width // LANES)) * LANES
        buf[r, pl.ds(c, LANES)] = zero16
        return carry

    lax.fori_loop(0, rows_n * (width // LANES), zstore, 0)


def _sc_gather_x(x, ei, ej, n_nodes, d):
    e = ei.shape[0]
    nw = NC * NS
    epw = e // nw
    nch = epw // CH
    npad = _npad(n_nodes)
    rpt = npad // NS

    @functools.partial(
        pl.kernel,
        out_type=jax.ShapeDtypeStruct((NC, npad, d), jnp.float32),
        mesh=_mesh(),
        compiler_params=pltpu.CompilerParams(use_tc_tiling_on_sc=True),
        scratch_types=[
            pltpu.VMEM_SHARED((npad, d), jnp.float32),
            [pltpu.VMEM((CH,), jnp.int32) for _ in range(8)],
            [pltpu.VMEM((CH,), jnp.int32) for _ in range(8)],
            [pltpu.VMEM((CH, d), jnp.float32) for _ in range(4)],
            [pltpu.SemaphoreType.DMA for _ in range(8)],
            [pltpu.SemaphoreType.DMA for _ in range(4)],
            [pltpu.SemaphoreType.DMA for _ in range(4)],
            pltpu.SemaphoreType.DMA,
        ],
    )
    def sc_fn(x_hbm, i_hbm, j_hbm, ax_out,
              ax_acc, ibuf, jbuf, rows, isem, gsem, osem, sem):
        cid = lax.axis_index("c")
        sid = lax.axis_index("s")
        wid = cid * NS + sid

        _zero_fill(rows[0], CH, d)
        r0 = sid * rpt
        nz = rpt // CH
        for z in range(nz):
            pltpu.async_copy(rows[0], ax_acc.at[pl.ds(r0 + z * CH, CH), :],
                             sem)
        for z in range(nz):
            pltpu.make_async_copy(
                rows[0], ax_acc.at[pl.ds(r0 + z * CH, CH), :], sem).wait()

        plsc.subcore_barrier()

        ebase = wid * epw

        def issue_idx(k, p8):
            pltpu.async_copy(i_hbm.at[pl.ds(ebase + k * CH, CH)], ibuf[p8],
                             isem[p8])
            pltpu.async_copy(j_hbm.at[pl.ds(ebase + k * CH, CH)], jbuf[p8],
                             isem[p8])

        def drain_idx(p8):
            pltpu.make_async_copy(i_hbm.at[pl.ds(0, CH)], ibuf[p8],
                                  isem[p8]).wait()
            pltpu.make_async_copy(j_hbm.at[pl.ds(0, CH)], jbuf[p8],
                                  isem[p8]).wait()

        def issue_in(p8, p4):
            pltpu.async_copy(x_hbm.at[jbuf[p8]], rows[p4], gsem[p4])

        def drain_in(p8, p4):
            pltpu.make_async_copy(x_hbm.at[jbuf[p8]], rows[p4],
                                  gsem[p4]).wait()

        def issue_out(p8, p4):
            pltpu.async_copy(rows[p4], ax_acc.at[ibuf[p8]], osem[p4],
                             add=True)

        def drain_out(p8, p4):
            pltpu.make_async_copy(rows[p4], ax_acc.at[ibuf[p8]],
                                  osem[p4]).wait()

        for k0 in range(4):
            issue_idx(k0, k0)
        drain_idx(0)
        issue_in(0, 0)
        drain_idx(1)
        issue_in(1, 1)

        def step(it, carry):
            for b8 in range(8):
                k = it * 8 + b8
                b4 = b8 % 4

                @pl.when(k < nch)
                def _():
                    drain_in(b8, b4)
                    issue_out(b8, b4)

                    @pl.when(k >= 2)
                    def _():
                        drain_out((b8 - 2) % 8, (b4 - 2) % 4)

                    @pl.when(k + 2 < nch)
                    def _():
                        drain_idx((b8 + 2) % 8)
                        issue_in((b8 + 2) % 8, (b4 + 2) % 4)

                    @pl.when(k + 4 < nch)
                    def _():
                        issue_idx(k + 4, (b8 + 4) % 8)
            return carry

        lax.fori_loop(0, -(-nch // 8), step, 0)

        for kk in (nch - 2, nch - 1):
            drain_out(kk % 8, kk % 4)

        plsc.subcore_barrier()

        pltpu.async_copy(ax_acc.at[pl.ds(r0, rpt), :],
                         ax_out.at[cid, pl.ds(r0, rpt), :], sem)
        pltpu.make_async_copy(ax_acc.at[pl.ds(r0, rpt), :],
                              ax_out.at[cid, pl.ds(r0, rpt), :], sem).wait()

    return sc_fn(x, ei, ej)


SDW = 32


def _sc_attr(ei, attr, n_nodes):
    e = ei.shape[0]
    ed = attr.shape[1]
    nw = NC * NS
    epw = e // nw
    nch = epw // CH
    npad = _npad(n_nodes)
    rpt = npad // NS

    @functools.partial(
        pl.kernel,
        out_type=jax.ShapeDtypeStruct((NC, npad, SDW), jnp.float32),
        mesh=_mesh(),
        compiler_params=pltpu.CompilerParams(use_tc_tiling_on_sc=False),
        scratch_types=[
            pltpu.VMEM_SHARED((npad, SDW), jnp.float32),
            [pltpu.VMEM((CH,), jnp.int32) for _ in range(8)],
            [pltpu.VMEM((CH, SDW), jnp.float32) for _ in range(4)],
            [pltpu.SemaphoreType.DMA for _ in range(8)],
            [pltpu.SemaphoreType.DMA for _ in range(4)],
            [pltpu.SemaphoreType.DMA for _ in range(4)],
            pltpu.SemaphoreType.DMA,
        ],
    )
    def sc_fn(i_hbm, attr_hbm, sd_out,
              sd_acc, ibuf, attrb, isem, gsem, osem, sem):
        cid = lax.axis_index("c")
        sid = lax.axis_index("s")
        wid = cid * NS + sid

        _zero_fill(attrb[0], CH, SDW)

        r0 = sid * rpt
        nz = rpt // CH
        for z in range(nz):
            pltpu.async_copy(attrb[0], sd_acc.at[pl.ds(r0 + z * CH, CH), :],
                             sem)
        for z in range(nz):
            pltpu.make_async_copy(
                attrb[0], sd_acc.at[pl.ds(r0 + z * CH, CH), :], sem).wait()

        onehot = jnp.where(
            lax.broadcasted_iota(jnp.int32, (LANES,), 0) == 0, 1.0, 0.0
        ).astype(jnp.float32)
        for p in range(4):
            def ostore(r, carry, _p=p):
                attrb[_p][r, pl.ds(ed, LANES)] = onehot
                return carry

            lax.fori_loop(0, CH, ostore, 0)

        plsc.subcore_barrier()

        ebase = wid * epw

        def issue_idx(k, p8):
            pltpu.async_copy(i_hbm.at[pl.ds(ebase + k * CH, CH)], ibuf[p8],
                             isem[p8])

        def drain_idx(p8):
            pltpu.make_async_copy(i_hbm.at[pl.ds(0, CH)], ibuf[p8],
                                  isem[p8]).wait()

        def issue_in(k, p4):
            pltpu.async_copy(attr_hbm.at[pl.ds(ebase + k * CH, CH), :],
                             attrb[p4].at[:, pl.ds(0, ed)], gsem[p4])

        def drain_in(p4):
            pltpu.make_async_copy(attr_hbm.at[pl.ds(0, CH), :],
                                  attrb[p4].at[:, pl.ds(0, ed)],
                                  gsem[p4]).wait()

        def issue_out(p8, p4):
            pltpu.async_copy(attrb[p4], sd_acc.at[ibuf[p8]], osem[p4],
                             add=True)

        def drain_out(p8, p4):
            pltpu.make_async_copy(attrb[p4], sd_acc.at[ibuf[p8]],
                                  osem[p4]).wait()

        for k0 in range(4):
            issue_idx(k0, k0)
        drain_idx(0)
        issue_in(0, 0)
        drain_idx(1)
        issue_in(1, 1)

        def step(it, carry):
            for b8 in range(8):
                k = it * 8 + b8
                b4 = b8 % 4

                @pl.when(k < nch)
                def _():
                    drain_in(b4)
                    issue_out(b8, b4)

                    @pl.when(k >= 2)
                    def _():
                        drain_out((b8 - 2) % 8, (b4 - 2) % 4)

                    @pl.when(k + 2 < nch)
                    def _():
                        drain_idx((b8 + 2) % 8)
                        issue_in(k + 2, (b4 + 2) % 4)

                    @pl.when(k + 4 < nch)
                    def _():
                        issue_idx(k + 4, (b8 + 4) % 8)
            return carry

        lax.fori_loop(0, -(-nch // 8), step, 0)

        for kk in (nch - 2, nch - 1):
            drain_out(kk % 8, kk % 4)

        plsc.subcore_barrier()

        pltpu.async_copy(sd_acc.at[pl.ds(r0, rpt), :],
                         sd_out.at[cid, pl.ds(r0, rpt), :], sem)
        pltpu.make_async_copy(sd_acc.at[pl.ds(r0, rpt), :],
                              sd_out.at[cid, pl.ds(r0, rpt), :], sem).wait()

    return sc_fn(ei, attr)


def _tc_dense(eps2, x, axp, sdp, wesd_t, w1_t, b1r, w2_t, b2r, g2, bt2):
    n, d = x.shape
    bn = 1000

    def body(eps_ref, x_ref, ax_ref, sd_ref, wesd_ref,
             w1_ref, b1_ref, w2_ref, b2_ref, g_ref, bt_ref, o_ref):
        ax = ax_ref[0] + ax_ref[1]
        sd = sd_ref[0] + sd_ref[1]
        agg = ax + jnp.dot(sd, wesd_ref[...],
                           preferred_element_type=jnp.float32)
        h = x_ref[...] * (1.0 + eps_ref[...]) + agg
        h = jnp.dot(h, w1_ref[...], preferred_element_type=jnp.float32)
        h = jnp.maximum(h + b1_ref[...], 0.0)
        h = jnp.dot(h, w2_ref[...], preferred_element_type=jnp.float32)
        h = h + b2_ref[...]
        mu = jnp.mean(h, axis=-1, keepdims=True)
        hc = h - mu
        var = jnp.mean(hc * hc, axis=-1, keepdims=True)
        o_ref[...] = hc * lax.rsqrt(var + 1e-5) * g_ref[...] + bt_ref[...]

    return pl.pallas_call(
        body,
        grid=(n // bn,),
        in_specs=[
            pl.BlockSpec((1, 1), lambda i: (0, 0)),
            pl.BlockSpec((bn, d), lambda i: (i, 0)),
            pl.BlockSpec((NC, bn, d), lambda i: (0, i, 0)),
            pl.BlockSpec((NC, bn, SDW), lambda i: (0, i, 0)),
            pl.BlockSpec((SDW, d), lambda i: (0, 0)),
            pl.BlockSpec((d, d), lambda i: (0, 0)),
            pl.BlockSpec((1, d), lambda i: (0, 0)),
            pl.BlockSpec((d, d), lambda i: (0, 0)),
            pl.BlockSpec((1, d), lambda i: (0, 0)),
            pl.BlockSpec((1, d), lambda i: (0, 0)),
            pl.BlockSpec((1, d), lambda i: (0, 0)),
        ],
        out_specs=pl.BlockSpec((bn, d), lambda i: (i, 0)),
        out_shape=jax.ShapeDtypeStruct((n, d), jnp.float32),
    )(eps2, x, axp, sdp, wesd_t, w1_t, b1r, w2_t, b2r, g2, bt2)


def kernel(x, edge_index, edge_attr, eps, We, be, W1, b1, W2, b2, gamma, beta):
    n, d = x.shape

    ei = edge_index[0]
    ej = edge_index[1]
    if ei.dtype != jnp.int32:
        ei = ei.astype(jnp.int32)
        ej = ej.astype(jnp.int32)
    attr = edge_attr
    if attr.dtype != jnp.float32:
        attr = attr.astype(jnp.float32)

    axp = _sc_gather_x(x, ei, ej, n, d)
    sdp = _sc_attr(ei, attr, n)

    ed = attr.shape[1]
    wesd_t = jnp.concatenate(
        [We.T, be[None, :], jnp.zeros((SDW - ed - 1, d), jnp.float32)],
        axis=0)
    eps2 = jnp.reshape(eps.astype(jnp.float32), (1, 1))
    return _tc_dense(eps2, x, axp, sdp, wesd_t,
                     W1.T, b1[None, :], W2.T, b2[None, :],
                     gamma[None, :], beta[None, :])


# speedup vs baseline: 1.0513x; 1.0513x over previous
import functools

import jax
import jax.numpy as jnp
from jax import lax
from jax.experimental import pallas as pl
from jax.experimental.pallas import tpu as pltpu
from jax.experimental.pallas import tpu_sc as plsc

NC = 2
NS = 16
LANES = 16
CH = 80


def _mesh():
    return plsc.VectorSubcoreMesh(core_axis_name="c", subcore_axis_name="s")


def _npad(n_nodes):
    return -(-n_nodes // (NS * CH)) * (NS * CH)


def _zero_fill(buf, rows_n, width):
    zero16 = jnp.zeros((LANES,), jnp.float32)

    def zstore(t, carry):
        r = t // (width // LANES)
        c = (t % (width // LANES)) * LANES
        buf[r, pl.ds(c, LANES)] = zero16
        return carry

    lax.fori_loop(0, rows_n * (width // LANES), zstore, 0)


def _sc_gather_x(x, ei, ej, n_nodes, d):
    e = ei.shape[0]
    nw = NC * NS
    epw = e // nw
    nch = epw // CH
    npad = _npad(n_nodes)
    rpt = npad // NS

    @functools.partial(
        pl.kernel,
        out_type=jax.ShapeDtypeStruct((NC, npad, d), jnp.float32),
        mesh=_mesh(),
        compiler_params=pltpu.CompilerParams(use_tc_tiling_on_sc=True),
        scratch_types=[
            pltpu.VMEM_SHARED((npad, d), jnp.float32),
            [pltpu.VMEM((CH,), jnp.int32) for _ in range(8)],
            [pltpu.VMEM((CH,), jnp.int32) for _ in range(8)],
            [pltpu.VMEM((CH, d), jnp.float32) for _ in range(4)],
            [pltpu.SemaphoreType.DMA for _ in range(8)],
            [pltpu.SemaphoreType.DMA for _ in range(4)],
            [pltpu.SemaphoreType.DMA for _ in range(4)],
            pltpu.SemaphoreType.DMA,
        ],
    )
    def sc_fn(x_hbm, i_hbm, j_hbm, ax_out,
              ax_acc, ibuf, jbuf, rows, isem, gsem, osem, sem):
        cid = lax.axis_index("c")
        sid = lax.axis_index("s")
        wid = cid * NS + sid

        _zero_fill(rows[0], CH, d)
        r0 = sid * rpt
        nz = rpt // CH
        for z in range(nz):
            pltpu.async_copy(rows[0], ax_acc.at[pl.ds(r0 + z * CH, CH), :],
                             sem)
        for z in range(nz):
            pltpu.make_async_copy(
                rows[0], ax_acc.at[pl.ds(r0 + z * CH, CH), :], sem).wait()

        plsc.subcore_barrier()

        ebase = wid * epw

        def issue_idx(k, p8):
            pltpu.async_copy(i_hbm.at[pl.ds(ebase + k * CH, CH)], ibuf[p8],
                             isem[p8])
            pltpu.async_copy(j_hbm.at[pl.ds(ebase + k * CH, CH)], jbuf[p8],
                             isem[p8])

        def drain_idx(p8):
            pltpu.make_async_copy(i_hbm.at[pl.ds(0, CH)], ibuf[p8],
                                  isem[p8]).wait()
            pltpu.make_async_copy(j_hbm.at[pl.ds(0, CH)], jbuf[p8],
                                  isem[p8]).wait()

        def issue_in(p8, p4):
            pltpu.async_copy(x_hbm.at[jbuf[p8]], rows[p4], gsem[p4])

        def drain_in(p8, p4):
            pltpu.make_async_copy(x_hbm.at[jbuf[p8]], rows[p4],
                                  gsem[p4]).wait()

        def issue_out(p8, p4):
            pltpu.async_copy(rows[p4], ax_acc.at[ibuf[p8]], osem[p4],
                             add=True)

        def drain_out(p8, p4):
            pltpu.make_async_copy(rows[p4], ax_acc.at[ibuf[p8]],
                                  osem[p4]).wait()

        for k0 in range(4):
            issue_idx(k0, k0)
        drain_idx(0)
        issue_in(0, 0)
        drain_idx(1)
        issue_in(1, 1)

        def step(it, carry):
            for b8 in range(8):
                k = it * 8 + b8
                b4 = b8 % 4

                @pl.when(k < nch)
                def _():
                    drain_in(b8, b4)
                    issue_out(b8, b4)

                    @pl.when(k >= 2)
                    def _():
                        drain_out((b8 - 2) % 8, (b4 - 2) % 4)

                    @pl.when(k + 2 < nch)
                    def _():
                        drain_idx((b8 + 2) % 8)
                        issue_in((b8 + 2) % 8, (b4 + 2) % 4)

                    @pl.when(k + 4 < nch)
                    def _():
                        issue_idx(k + 4, (b8 + 4) % 8)
            return carry

        lax.fori_loop(0, -(-nch // 8), step, 0)

        for kk in (nch - 2, nch - 1):
            drain_out(kk % 8, kk % 4)

        plsc.subcore_barrier()

        pltpu.async_copy(ax_acc.at[pl.ds(r0, rpt), :],
                         ax_out.at[cid, pl.ds(r0, rpt), :], sem)
        pltpu.make_async_copy(ax_acc.at[pl.ds(r0, rpt), :],
                              ax_out.at[cid, pl.ds(r0, rpt), :], sem).wait()

    return sc_fn(x, ei, ej)


def _sc_attr(ei, attr, n_nodes):
    e = ei.shape[0]
    ed = attr.shape[1]
    nw = NC * NS
    epw = e // nw
    nch = epw // CH
    npad = _npad(n_nodes)
    rpt = npad // NS

    @functools.partial(
        pl.kernel,
        out_type=(
            jax.ShapeDtypeStruct((NC, npad, ed), jnp.float32),
            jax.ShapeDtypeStruct((NC, npad, LANES), jnp.float32),
        ),
        mesh=_mesh(),
        compiler_params=pltpu.CompilerParams(use_tc_tiling_on_sc=False),
        scratch_types=[
            pltpu.VMEM_SHARED((npad, ed), jnp.float32),
            pltpu.VMEM_SHARED((npad, LANES), jnp.float32),
            [pltpu.VMEM((CH,), jnp.int32) for _ in range(8)],
            [pltpu.VMEM((CH, ed), jnp.float32) for _ in range(4)],
            pltpu.VMEM((CH, LANES), jnp.float32),
            [pltpu.SemaphoreType.DMA for _ in range(8)],
            [pltpu.SemaphoreType.DMA for _ in range(4)],
            [pltpu.SemaphoreType.DMA for _ in range(4)],
            pltpu.SemaphoreType.DMA,
        ],
    )
    def sc_fn(i_hbm, attr_hbm, s_out, deg_out,
              s_acc, deg_acc, ibuf, attrb, onesbuf, isem, gsem, osem, sem):
        cid = lax.axis_index("c")
        sid = lax.axis_index("s")
        wid = cid * NS + sid

        _zero_fill(attrb[0], CH, ed)
        ones16 = jnp.ones((LANES,), jnp.float32)

        def ostore(r, carry):
            onesbuf[r, pl.ds(0, LANES)] = ones16
            return carry

        lax.fori_loop(0, CH, ostore, 0)

        r0 = sid * rpt
        nz = rpt // CH
        for z in range(nz):
            pltpu.async_copy(attrb[0], s_acc.at[pl.ds(r0 + z * CH, CH), :],
                             sem)
            pltpu.async_copy(attrb[0], deg_acc.at[pl.ds(r0 + z * CH, CH), :],
                             sem)
        for z in range(nz):
            pltpu.make_async_copy(
                attrb[0], s_acc.at[pl.ds(r0 + z * CH, CH), :], sem).wait()
            pltpu.make_async_copy(
                attrb[0], deg_acc.at[pl.ds(r0 + z * CH, CH), :], sem).wait()

        plsc.subcore_barrier()

        ebase = wid * epw

        def issue_idx(k, p8):
            pltpu.async_copy(i_hbm.at[pl.ds(ebase + k * CH, CH)], ibuf[p8],
                             isem[p8])

        def drain_idx(p8):
            pltpu.make_async_copy(i_hbm.at[pl.ds(0, CH)], ibuf[p8],
                                  isem[p8]).wait()

        def issue_in(k, p4):
            pltpu.async_copy(attr_hbm.at[pl.ds(ebase + k * CH, CH), :],
                             attrb[p4], gsem[p4])

        def drain_in(p4):
            pltpu.make_async_copy(attr_hbm.at[pl.ds(0, CH), :], attrb[p4],
                                  gsem[p4]).wait()

        def issue_out(p8, p4):
            pltpu.async_copy(attrb[p4], s_acc.at[ibuf[p8]], osem[p4],
                             add=True)
            pltpu.async_copy(onesbuf, deg_acc.at[ibuf[p8]], osem[p4],
                             add=True)

        def drain_out(p8, p4):
            pltpu.make_async_copy(attrb[p4], s_acc.at[ibuf[p8]],
                                  osem[p4]).wait()
            pltpu.make_async_copy(onesbuf, deg_acc.at[ibuf[p8]],
                                  osem[p4]).wait()

        for k0 in range(4):
            issue_idx(k0, k0)
        drain_idx(0)
        issue_in(0, 0)
        drain_idx(1)
        issue_in(1, 1)

        def step(it, carry):
            for b8 in range(8):
                k = it * 8 + b8
                b4 = b8 % 4

                @pl.when(k < nch)
                def _():
                    drain_in(b4)
                    issue_out(b8, b4)

                    @pl.when(k >= 2)
                    def _():
                        drain_out((b8 - 2) % 8, (b4 - 2) % 4)

                    @pl.when(k + 2 < nch)
                    def _():
                        drain_idx((b8 + 2) % 8)
                        issue_in(k + 2, (b4 + 2) % 4)

                    @pl.when(k + 4 < nch)
                    def _():
                        issue_idx(k + 4, (b8 + 4) % 8)
            return carry

        lax.fori_loop(0, -(-nch // 8), step, 0)

        for kk in (nch - 2, nch - 1):
            drain_out(kk % 8, kk % 4)

        plsc.subcore_barrier()

        pltpu.async_copy(s_acc.at[pl.ds(r0, rpt), :],
                         s_out.at[cid, pl.ds(r0, rpt), :], sem)
        pltpu.async_copy(deg_acc.at[pl.ds(r0, rpt), :],
                         deg_out.at[cid, pl.ds(r0, rpt), :], sem)
        pltpu.make_async_copy(s_acc.at[pl.ds(r0, rpt), :],
                              s_out.at[cid, pl.ds(r0, rpt), :], sem).wait()
        pltpu.make_async_copy(deg_acc.at[pl.ds(r0, rpt), :],
                              deg_out.at[cid, pl.ds(r0, rpt), :], sem).wait()

    return sc_fn(ei, attr)


def _tc_dense(eps2, x, axp, sp, degp, we_t, be_r, w1_t, b1r, w2_t, b2r,
              g2, bt2):
    n, d = x.shape
    ed = we_t.shape[0]
    bn = 1000

    def body(eps_ref, x_ref, ax_ref, s_ref, deg_ref, we_ref, be_ref,
             w1_ref, b1_ref, w2_ref, b2_ref, g_ref, bt_ref, o_ref):
        ax = ax_ref[0] + ax_ref[1]
        s = s_ref[0] + s_ref[1]
        deg = (deg_ref[0] + deg_ref[1])[:, 0:1]
        agg = ax + jnp.dot(s, we_ref[...], preferred_element_type=jnp.float32)
        agg = agg + deg * be_ref[...]
        h = x_ref[...] * (1.0 + eps_ref[...]) + agg
        h = jnp.dot(h, w1_ref[...], preferred_element_type=jnp.float32)
        h = jnp.maximum(h + b1_ref[...], 0.0)
        h = jnp.dot(h, w2_ref[...], preferred_element_type=jnp.float32)
        h = h + b2_ref[...]
        mu = jnp.mean(h, axis=-1, keepdims=True)
        hc = h - mu
        var = jnp.mean(hc * hc, axis=-1, keepdims=True)
        o_ref[...] = hc * lax.rsqrt(var + 1e-5) * g_ref[...] + bt_ref[...]

    return pl.pallas_call(
        body,
        grid=(n // bn,),
        in_specs=[
            pl.BlockSpec((1, 1), lambda i: (0, 0)),
            pl.BlockSpec((bn, d), lambda i: (i, 0)),
            pl.BlockSpec((NC, bn, d), lambda i: (0, i, 0)),
            pl.BlockSpec((NC, bn, ed), lambda i: (0, i, 0)),
            pl.BlockSpec((NC, bn, LANES), lambda i: (0, i, 0)),
            pl.BlockSpec((ed, d), lambda i: (0, 0)),
            pl.BlockSpec((1, d), lambda i: (0, 0)),
            pl.BlockSpec((d, d), lambda i: (0, 0)),
            pl.BlockSpec((1, d), lambda i: (0, 0)),
            pl.BlockSpec((d, d), lambda i: (0, 0)),
            pl.BlockSpec((1, d), lambda i: (0, 0)),
            pl.BlockSpec((1, d), lambda i: (0, 0)),
            pl.BlockSpec((1, d), lambda i: (0, 0)),
        ],
        out_specs=pl.BlockSpec((bn, d), lambda i: (i, 0)),
        out_shape=jax.ShapeDtypeStruct((n, d), jnp.float32),
    )(eps2, x, axp, sp, degp, we_t, be_r, w1_t, b1r, w2_t, b2r, g2, bt2)


def kernel(x, edge_index, edge_attr, eps, We, be, W1, b1, W2, b2, gamma, beta):
    n, d = x.shape

    ei = edge_index[0]
    ej = edge_index[1]
    if ei.dtype != jnp.int32:
        ei = ei.astype(jnp.int32)
        ej = ej.astype(jnp.int32)
    attr = edge_attr
    if attr.dtype != jnp.float32:
        attr = attr.astype(jnp.float32)

    axp = _sc_gather_x(x, ei, ej, n, d)
    sp, degp = _sc_attr(ei, attr, n)

    eps2 = jnp.reshape(eps.astype(jnp.float32), (1, 1))
    return _tc_dense(eps2, x, axp, sp, degp, We.T, be[None, :],
                     W1.T, b1[None, :], W2.T, b2[None, :],
                     gamma[None, :], beta[None, :])
